# Initial kernel scaffold; baseline (speedup 1.0000x reference)
import jax
import jax.numpy as jnp
from jax.experimental import pallas as pl


def kernel(y, bow_genre_w, artist_w, total_w, abs_w, rel_w):
    raise NotImplementedError("write your pallas kernel here")


# speedup vs baseline: 4.4618x; 4.4618x over previous
import functools

import jax
import jax.numpy as jnp
from jax import lax
from jax.experimental import pallas as pl
from jax.experimental.pallas import tpu as pltpu
from jax.experimental.pallas import tpu_sc as plsc

SR = 44100
MIN_DUR = 24
MAX_DUR = 600
N_TIME = 2048
T_BINS = 128
OUT_W = 64
TOT_MIN = float(MIN_DUR * SR)
TOT_MAX = float(MAX_DUR * SR)

TB = 512


def _start_emb_sc(aidx, gidx, artist_w, bow_aug, n, k, w):
    info = plsc.get_sparse_core_info()
    nw = info.num_cores * info.num_subcores
    b = n // nw
    mesh = plsc.VectorSubcoreMesh(core_axis_name="c", subcore_axis_name="s")

    @functools.partial(
        pl.kernel,
        mesh=mesh,
        out_type=jax.ShapeDtypeStruct((n, w), jnp.float32),
        scratch_types=[
            pltpu.VMEM((b,), jnp.int32),
            pltpu.VMEM((b * k,), jnp.int32),
            pltpu.VMEM((b, w), jnp.float32),
            pltpu.VMEM((b * k, w), jnp.float32),
            pltpu.VMEM((b, w), jnp.float32),
            pltpu.SemaphoreType.DMA,
        ],
        compiler_params=pltpu.CompilerParams(use_tc_tiling_on_sc=False),
    )
    def sc_kernel(aidx_hbm, gidx_hbm, aw_hbm, bw_hbm, out_hbm,
                  aidx_v, gidx_v, arows, grows, acc, sem):
        wid = lax.axis_index("s") * info.num_cores + lax.axis_index("c")
        pltpu.sync_copy(aidx_hbm.at[pl.ds(wid * b, b)], aidx_v)
        pltpu.sync_copy(gidx_hbm.at[pl.ds(wid * b * k, b * k)], gidx_v)
        pltpu.async_copy(aw_hbm.at[aidx_v], arows, sem).wait()
        pltpu.async_copy(bw_hbm.at[gidx_v], grows, sem).wait()
        for i in range(b):
            for wv in range(w // 16):
                s = arows[i, pl.ds(wv * 16, 16)]
                for j in range(k):
                    s = s + grows[i * k + j, pl.ds(wv * 16, 16)]
                acc[i, pl.ds(wv * 16, 16)] = s
        pltpu.sync_copy(acc, out_hbm.at[pl.ds(wid * b, b)])

    return sc_kernel(aidx, gidx, artist_w, bow_aug)


def _pos_body(y_ref, tw_ref, aw_ref, rw_ref, o_ref):
    n = pl.program_id(0)
    tb = pl.program_id(1)
    tot = y_ref[n, 0].astype(jnp.float32)
    off_i = y_ref[n, 1]
    len_i = y_ref[n, 2]
    start = off_i.astype(jnp.float32)
    end = (off_i + len_i).astype(jnp.float32)

    t0 = (tb * TB).astype(jnp.float32)
    interp = (lax.broadcasted_iota(jnp.int32, (TB, 1), 0).astype(jnp.float32)
              + t0) / float(N_TIME)
    lanes = lax.broadcasted_iota(jnp.int32, (TB, T_BINS), 1)

    pos_a = start + (end - start) * interp
    bins_a = jnp.clip(jnp.floor(T_BINS * (pos_a / TOT_MAX)).astype(jnp.int32),
                      0, T_BINS - 1)
    oh_a = (lanes == bins_a).astype(jnp.float32)
    emb_a = jnp.dot(oh_a, aw_ref[...], preferred_element_type=jnp.float32)

    ps_r = start / tot
    pe_r = jnp.clip(end / tot, 0.0, 1.0)
    pos_r = ps_r + (pe_r - ps_r) * interp
    bins_r = jnp.clip(jnp.floor(T_BINS * pos_r).astype(jnp.int32), 0, T_BINS - 1)
    oh_r = (lanes == bins_r).astype(jnp.float32)
    emb_r = jnp.dot(oh_r, rw_ref[...], preferred_element_type=jnp.float32)

    norm_t = (tot - TOT_MIN) / (TOT_MAX - TOT_MIN)
    bin_t = jnp.clip(jnp.floor(T_BINS * norm_t).astype(jnp.int32), 0, T_BINS - 1)
    oh_t = (lax.broadcasted_iota(jnp.int32, (1, T_BINS), 1) == bin_t).astype(
        jnp.float32)
    emb_t = jnp.dot(oh_t, tw_ref[...], preferred_element_type=jnp.float32)

    o_ref[0] = emb_a + emb_r + emb_t


def _pos_emb_tc(y, total_w, abs_w, rel_w, n, w, interpret=False):
    grid = (n, N_TIME // TB)
    return pl.pallas_call(
        _pos_body,
        grid=grid,
        in_specs=[
            pl.BlockSpec(memory_space=pltpu.SMEM),
            pl.BlockSpec((T_BINS, w), lambda i, j: (0, 0)),
            pl.BlockSpec((T_BINS, w), lambda i, j: (0, 0)),
            pl.BlockSpec((T_BINS, w), lambda i, j: (0, 0)),
        ],
        out_specs=pl.BlockSpec((1, TB, w), lambda i, j: (i, j, 0)),
        out_shape=jax.ShapeDtypeStruct((n, N_TIME, w), jnp.float32),
        compiler_params=pltpu.CompilerParams(
            dimension_semantics=("parallel", "parallel")),
        interpret=interpret,
    )(y, total_w, abs_w, rel_w)


def kernel(y, bow_genre_w, artist_w, total_w, abs_w, rel_w):
    n = y.shape[0]
    k = y.shape[1] - 4
    g = bow_genre_w.shape[0]
    w = bow_genre_w.shape[1]

    artist_idx = y[:, 3].astype(jnp.int32)
    genre = y[:, 4:]
    gidx = jnp.where(genre < 0, g, genre).astype(jnp.int32).reshape(-1)
    bow_aug = jnp.concatenate(
        [bow_genre_w, jnp.zeros((1, w), jnp.float32)], axis=0)

    start2d = _start_emb_sc(artist_idx, gidx, artist_w, bow_aug, n, k, w)
    start_emb = start2d.reshape(n, 1, w)
    pos_emb = _pos_emb_tc(y, total_w, abs_w, rel_w, n, w)
    return (start_emb, pos_emb)


# speedup vs baseline: 8.0795x; 1.8108x over previous
import functools

import jax
import jax.numpy as jnp
from jax import lax
from jax.experimental import pallas as pl
from jax.experimental.pallas import tpu as pltpu
from jax.experimental.pallas import tpu_sc as plsc

SR = 44100
MIN_DUR = 24
MAX_DUR = 600
N_TIME = 2048
T_BINS = 128
OUT_W = 64
TOT_MIN = float(MIN_DUR * SR)
TOT_MAX = float(MAX_DUR * SR)

TB = 2048


def _start_emb_sc(aidx, gidx, artist_w, bow_aug, n, k, w):
    info = plsc.get_sparse_core_info()
    nw = info.num_cores * info.num_subcores
    b = n // nw
    mesh = plsc.VectorSubcoreMesh(core_axis_name="c", subcore_axis_name="s")

    @functools.partial(
        pl.kernel,
        mesh=mesh,
        out_type=jax.ShapeDtypeStruct((n, w), jnp.float32),
        scratch_types=[
            pltpu.VMEM((b,), jnp.int32),
            pltpu.VMEM((b * k,), jnp.int32),
            pltpu.VMEM((b, w), jnp.float32),
            pltpu.VMEM((b * k, w), jnp.float32),
            pltpu.VMEM((b, w), jnp.float32),
            pltpu.SemaphoreType.DMA,
        ],
        compiler_params=pltpu.CompilerParams(use_tc_tiling_on_sc=False),
    )
    def sc_kernel(aidx_hbm, gidx_hbm, aw_hbm, bw_hbm, out_hbm,
                  aidx_v, gidx_v, arows, grows, acc, sem):
        wid = lax.axis_index("s") * info.num_cores + lax.axis_index("c")
        pltpu.sync_copy(aidx_hbm.at[pl.ds(wid * b, b)], aidx_v)
        pltpu.sync_copy(gidx_hbm.at[pl.ds(wid * b * k, b * k)], gidx_v)
        pltpu.async_copy(aw_hbm.at[aidx_v], arows, sem).wait()
        pltpu.async_copy(bw_hbm.at[gidx_v], grows, sem).wait()
        for i in range(b):
            for wv in range(w // 16):
                s = arows[i, pl.ds(wv * 16, 16)]
                for j in range(k):
                    s = s + grows[i * k + j, pl.ds(wv * 16, 16)]
                acc[i, pl.ds(wv * 16, 16)] = s
        pltpu.sync_copy(acc, out_hbm.at[pl.ds(wid * b, b)])

    return sc_kernel(aidx, gidx, artist_w, bow_aug)


def _pos_body(y_ref, dw_ref, dwt_ref, o_ref):
    n = pl.program_id(0)
    tot = y_ref[n, 0].astype(jnp.float32)
    off_i = y_ref[n, 1]
    len_i = y_ref[n, 2]
    start = off_i.astype(jnp.float32)
    end = (off_i + len_i).astype(jnp.float32)

    tiny = jnp.float32(1e-30)
    lanes = lax.broadcasted_iota(
        jnp.int32, (1, T_BINS), 1).astype(jnp.float32)
    thr_a = ((lanes * jnp.float32(TOT_MAX / T_BINS) - start)
             * (float(N_TIME) / jnp.maximum(end - start, tiny)))
    ps_r = start / tot
    pe_r = jnp.clip(end / tot, 0.0, 1.0)
    thr_r = ((lanes * jnp.float32(1.0 / T_BINS) - ps_r)
             * (float(N_TIME) / jnp.maximum(pe_r - ps_r, tiny)))
    thr_ai = jnp.ceil(thr_a).astype(jnp.int32)
    thr_ri = jnp.ceil(thr_r).astype(jnp.int32)

    t_i = lax.broadcasted_iota(jnp.int32, (TB, 1), 0)
    steps_a = (t_i >= thr_ai).astype(jnp.float32)
    steps_r = (t_i >= thr_ri).astype(jnp.float32)
    steps = jnp.concatenate([steps_a, steps_r], axis=1)
    emb = jnp.dot(steps, dw_ref[...], preferred_element_type=jnp.float32)

    xt = (tot - TOT_MIN) * jnp.float32(T_BINS / (TOT_MAX - TOT_MIN))
    st = (xt >= lanes).astype(jnp.float32)
    embt = jnp.dot(st, dwt_ref[...], preferred_element_type=jnp.float32)

    o_ref[0] = emb + embt


def _telescope(w):
    return jnp.concatenate([w[:1], w[1:] - w[:-1]], axis=0)


def _pos_emb_tc(y, total_w, abs_w, rel_w, n, w, interpret=False):
    dw = jnp.concatenate([_telescope(abs_w), _telescope(rel_w)], axis=0)
    dwt = _telescope(total_w)
    grid = (n, N_TIME // TB)
    return pl.pallas_call(
        _pos_body,
        grid=grid,
        in_specs=[
            pl.BlockSpec(memory_space=pltpu.SMEM),
            pl.BlockSpec((2 * T_BINS, w), lambda i, j: (0, 0)),
            pl.BlockSpec((T_BINS, w), lambda i, j: (0, 0)),
        ],
        out_specs=pl.BlockSpec((1, TB, w), lambda i, j: (i, j, 0)),
        out_shape=jax.ShapeDtypeStruct((n, N_TIME, w), jnp.float32),
        compiler_params=pltpu.CompilerParams(
            dimension_semantics=("parallel", "parallel")),
        interpret=interpret,
    )(y, dw, dwt)


def kernel(y, bow_genre_w, artist_w, total_w, abs_w, rel_w):
    n = y.shape[0]
    k = y.shape[1] - 4
    g = bow_genre_w.shape[0]
    w = bow_genre_w.shape[1]

    artist_idx = y[:, 3].astype(jnp.int32)
    genre = y[:, 4:]
    gidx = jnp.where(genre < 0, g, genre).astype(jnp.int32).reshape(-1)
    bow_aug = jnp.concatenate(
        [bow_genre_w, jnp.zeros((1, w), jnp.float32)], axis=0)

    start2d = _start_emb_sc(artist_idx, gidx, artist_w, bow_aug, n, k, w)
    start_emb = start2d.reshape(n, 1, w)
    pos_emb = _pos_emb_tc(y, total_w, abs_w, rel_w, n, w)
    return (start_emb, pos_emb)
